# baseline (speedup 1.0000x reference)
import functools

import jax
import jax.numpy as jnp
from jax import lax
from jax.experimental import pallas as pl
from jax.experimental.pallas import tpu as pltpu

K = 16
N = 16
P = 2048
C = 64
EDGES = N * P * K


def _stats_kernel(x_ref, s_ref, ss_ref):
    i = pl.program_id(0)
    x = x_ref[...]
    s = jnp.sum(x, axis=0, keepdims=True)
    ss = jnp.sum(x * x, axis=0, keepdims=True)

    @pl.when(i == 0)
    def _():
        s_ref[...] = s
        ss_ref[...] = ss

    @pl.when(i > 0)
    def _():
        s_ref[...] += s
        ss_ref[...] += ss


def _stats(x, rows_per_block):
    rows = x.shape[0]
    grid = rows // rows_per_block
    s, ss = pl.pallas_call(
        _stats_kernel,
        grid=(grid,),
        in_specs=[pl.BlockSpec((rows_per_block, 64), lambda i: (i, 0))],
        out_specs=[
            pl.BlockSpec((1, 64), lambda i: (0, 0)),
            pl.BlockSpec((1, 64), lambda i: (0, 0)),
        ],
        out_shape=[
            jax.ShapeDtypeStruct((1, 64), jnp.float32),
            jax.ShapeDtypeStruct((1, 64), jnp.float32),
        ],
    )(x)
    return s[0], ss[0]


def _mid_kernel(y0_ref, a0_ref, c0_ref, w1_ref, y1_ref, s_ref, ss_ref):
    i = pl.program_id(0)
    e0 = jnp.maximum(y0_ref[...] * a0_ref[...] + c0_ref[...], 0.0)
    y1 = jnp.dot(e0, w1_ref[...], preferred_element_type=jnp.float32)
    y1_ref[...] = y1
    s = jnp.sum(y1, axis=0, keepdims=True)
    ss = jnp.sum(y1 * y1, axis=0, keepdims=True)

    @pl.when(i == 0)
    def _():
        s_ref[...] = s
        ss_ref[...] = ss

    @pl.when(i > 0)
    def _():
        s_ref[...] += s
        ss_ref[...] += ss


def _mid(y0, a0, c0, w1t, rows_per_block):
    rows = y0.shape[0]
    grid = rows // rows_per_block
    y1, s, ss = pl.pallas_call(
        _mid_kernel,
        grid=(grid,),
        in_specs=[
            pl.BlockSpec((rows_per_block, 64), lambda i: (i, 0)),
            pl.BlockSpec((1, 64), lambda i: (0, 0)),
            pl.BlockSpec((1, 64), lambda i: (0, 0)),
            pl.BlockSpec((64, 64), lambda i: (0, 0)),
        ],
        out_specs=[
            pl.BlockSpec((rows_per_block, 64), lambda i: (i, 0)),
            pl.BlockSpec((1, 64), lambda i: (0, 0)),
            pl.BlockSpec((1, 64), lambda i: (0, 0)),
        ],
        out_shape=[
            jax.ShapeDtypeStruct((rows, 64), jnp.float32),
            jax.ShapeDtypeStruct((1, 64), jnp.float32),
            jax.ShapeDtypeStruct((1, 64), jnp.float32),
        ],
    )(y0, a0[None, :], c0[None, :], w1t)
    return y1, s[0], ss[0]


def _final_kernel(y1_ref, a1_ref, c1_ref, sc_ref, out_ref):
    e1 = jnp.maximum(y1_ref[...] * a1_ref[...] + c1_ref[...], 0.0)
    rows = e1.shape[0]
    h = jnp.mean(e1.reshape(rows // K, K, 64), axis=1)
    out_ref[...] = jnp.maximum(h + sc_ref[...], 0.0)


def _final(y1, a1, c1, sc, rows_per_block):
    rows = y1.shape[0]
    grid = rows // rows_per_block
    out = pl.pallas_call(
        _final_kernel,
        grid=(grid,),
        in_specs=[
            pl.BlockSpec((rows_per_block, 64), lambda i: (i, 0)),
            pl.BlockSpec((1, 64), lambda i: (0, 0)),
            pl.BlockSpec((1, 64), lambda i: (0, 0)),
            pl.BlockSpec((rows_per_block // K, 64), lambda i: (i, 0)),
        ],
        out_specs=pl.BlockSpec((rows_per_block // K, 64), lambda i: (i, 0)),
        out_shape=jax.ShapeDtypeStruct((rows // K, 64), jnp.float32),
    )(y1, a1[None, :], c1[None, :], sc)
    return out


def _bn_coeffs(s, ss, count, g, be):
    m = s / count
    v = ss / count - m * m
    a = g * lax.rsqrt(v + 1e-5)
    c = be - m * a
    return a, c


def kernel(features, W0, b0, g0, be0, W1, b1, g1, be1, Ws, bs, gs, bes):
    pts = features[:, :, 0:2]
    X = features[:, :, 2:]

    rA = jnp.sum(pts * pts, axis=2, keepdims=True)
    dist = rA - 2.0 * jnp.matmul(pts, jnp.transpose(pts, (0, 2, 1))) + jnp.transpose(rA, (0, 2, 1))
    _, idx = jax.lax.top_k(-dist, K + 1)
    idx = idx[:, :, 1:]

    W0a = W0[:, :C]
    W0b = W0[:, C:]
    U = jnp.einsum('npc,oc->npo', X, W0b)
    V = jnp.einsum('npc,oc->npo', X, W0a + W0b) + b0[None, None, :]

    G = jnp.take_along_axis(U, idx.reshape(N, P * K)[:, :, None], axis=1).reshape(N, P, K, C)
    y0 = (V[:, :, None, :] - G).reshape(EDGES, C)

    s0, ss0 = _stats(y0, 8192)
    a0, c0 = _bn_coeffs(s0, ss0, float(EDGES), g0, be0)
    y1, s1, ss1 = _mid(y0, a0, c0, W1.T, 8192)
    a1, c1 = _bn_coeffs(s1, ss1, float(EDGES), g1, be1)

    ysc = (jnp.einsum('npc,oc->npo', X, Ws) + bs[None, None, :]).reshape(N * P, C)
    ssc, sssc = _stats(ysc, 4096)
    asc, csc = _bn_coeffs(ssc, sssc, float(N * P), gs, bes)
    sc = ysc * asc[None, :] + csc[None, :]

    out = _final(y1, a1, c1, sc, 8192)
    return jnp.transpose(out.reshape(N, P, C), (0, 2, 1))


# speedup vs baseline: 4.9116x; 4.9116x over previous
import functools

import jax
import jax.numpy as jnp
from jax import lax
from jax.experimental import pallas as pl
from jax.experimental.pallas import tpu as pltpu

K = 16
N = 16
P = 2048
C = 64
EDGES = N * P * K
RT = 256
IMAX = 2147483647


def _knn_kernel(pts_r_ref, ptsT_ref, out_ref):
    n = pl.program_id(0)
    pr = pts_r_ref[0]
    pcT = ptsT_ref[0]
    rr = jnp.sum(pr * pr, axis=1, keepdims=True)
    rc = jnp.sum(pcT * pcT, axis=0, keepdims=True)
    dot = jnp.dot(pr, pcT, preferred_element_type=jnp.float32)
    dist = jnp.maximum(rr - 2.0 * dot + rc, 0.0)
    col = lax.broadcasted_iota(jnp.int32, (RT, P), 1)
    bits = lax.bitcast_convert_type(dist, jnp.int32)
    keys0 = jnp.bitwise_or(jnp.bitwise_and(bits, jnp.int32(-2048)), col)

    def body(k, carry):
        keys, acc = carry
        m = keys[:, 0:128]
        for j in range(1, 16):
            m = jnp.minimum(m, keys[:, j * 128:(j + 1) * 128])
        g = jnp.min(m, axis=1, keepdims=True)
        keys = jnp.where(keys == g, IMAX, keys)
        lane = lax.broadcasted_iota(jnp.int32, (RT, K + 1), 1)
        acc = jnp.where(lane == k, jnp.bitwise_and(g, jnp.int32(2047)), acc)
        return keys, acc

    _, acc = lax.fori_loop(0, K + 1, body,
                           (keys0, jnp.zeros((RT, K + 1), jnp.int32)))
    out_ref[0] = acc[:, 1:] + n * P


def _knn(pts):
    ptsT = jnp.transpose(pts, (0, 2, 1))
    return pl.pallas_call(
        _knn_kernel,
        grid=(N, P // RT),
        in_specs=[
            pl.BlockSpec((1, RT, 2), lambda n, r: (n, r, 0)),
            pl.BlockSpec((1, 2, P), lambda n, r: (n, 0, 0)),
        ],
        out_specs=pl.BlockSpec((1, RT, K), lambda n, r: (n, r, 0)),
        out_shape=jax.ShapeDtypeStruct((N, P, K), jnp.int32),
    )(pts, ptsT)


def _stats_kernel(x_ref, s_ref, ss_ref):
    i = pl.program_id(0)
    x = x_ref[...]
    s = jnp.sum(x, axis=0, keepdims=True)
    ss = jnp.sum(x * x, axis=0, keepdims=True)

    @pl.when(i == 0)
    def _():
        s_ref[...] = s
        ss_ref[...] = ss

    @pl.when(i > 0)
    def _():
        s_ref[...] += s
        ss_ref[...] += ss


def _stats(x, rows_per_block):
    rows = x.shape[0]
    grid = rows // rows_per_block
    s, ss = pl.pallas_call(
        _stats_kernel,
        grid=(grid,),
        in_specs=[pl.BlockSpec((rows_per_block, 64), lambda i: (i, 0))],
        out_specs=[
            pl.BlockSpec((1, 64), lambda i: (0, 0)),
            pl.BlockSpec((1, 64), lambda i: (0, 0)),
        ],
        out_shape=[
            jax.ShapeDtypeStruct((1, 64), jnp.float32),
            jax.ShapeDtypeStruct((1, 64), jnp.float32),
        ],
    )(x)
    return s[0], ss[0]


def _mid_kernel(y0_ref, a0_ref, c0_ref, w1_ref, y1_ref, s_ref, ss_ref):
    i = pl.program_id(0)
    e0 = jnp.maximum(y0_ref[...] * a0_ref[...] + c0_ref[...], 0.0)
    y1 = jnp.dot(e0, w1_ref[...], preferred_element_type=jnp.float32)
    y1_ref[...] = y1
    s = jnp.sum(y1, axis=0, keepdims=True)
    ss = jnp.sum(y1 * y1, axis=0, keepdims=True)

    @pl.when(i == 0)
    def _():
        s_ref[...] = s
        ss_ref[...] = ss

    @pl.when(i > 0)
    def _():
        s_ref[...] += s
        ss_ref[...] += ss


def _mid(y0, a0, c0, w1t, rows_per_block):
    rows = y0.shape[0]
    grid = rows // rows_per_block
    y1, s, ss = pl.pallas_call(
        _mid_kernel,
        grid=(grid,),
        in_specs=[
            pl.BlockSpec((rows_per_block, 64), lambda i: (i, 0)),
            pl.BlockSpec((1, 64), lambda i: (0, 0)),
            pl.BlockSpec((1, 64), lambda i: (0, 0)),
            pl.BlockSpec((64, 64), lambda i: (0, 0)),
        ],
        out_specs=[
            pl.BlockSpec((rows_per_block, 64), lambda i: (i, 0)),
            pl.BlockSpec((1, 64), lambda i: (0, 0)),
            pl.BlockSpec((1, 64), lambda i: (0, 0)),
        ],
        out_shape=[
            jax.ShapeDtypeStruct((rows, 64), jnp.float32),
            jax.ShapeDtypeStruct((1, 64), jnp.float32),
            jax.ShapeDtypeStruct((1, 64), jnp.float32),
        ],
    )(y0, a0[None, :], c0[None, :], w1t)
    return y1, s[0], ss[0]


def _final_kernel(y1_ref, a1_ref, c1_ref, sc_ref, out_ref):
    e1 = jnp.maximum(y1_ref[...] * a1_ref[...] + c1_ref[...], 0.0)
    rows = e1.shape[0]
    h = jnp.mean(e1.reshape(rows // K, K, 64), axis=1)
    out_ref[...] = jnp.maximum(h + sc_ref[...], 0.0)


def _final(y1, a1, c1, sc, rows_per_block):
    rows = y1.shape[0]
    grid = rows // rows_per_block
    out = pl.pallas_call(
        _final_kernel,
        grid=(grid,),
        in_specs=[
            pl.BlockSpec((rows_per_block, 64), lambda i: (i, 0)),
            pl.BlockSpec((1, 64), lambda i: (0, 0)),
            pl.BlockSpec((1, 64), lambda i: (0, 0)),
            pl.BlockSpec((rows_per_block // K, 64), lambda i: (i, 0)),
        ],
        out_specs=pl.BlockSpec((rows_per_block // K, 64), lambda i: (i, 0)),
        out_shape=jax.ShapeDtypeStruct((rows // K, 64), jnp.float32),
    )(y1, a1[None, :], c1[None, :], sc)
    return out


def _bn_coeffs(s, ss, count, g, be):
    m = s / count
    v = ss / count - m * m
    a = g * lax.rsqrt(v + 1e-5)
    c = be - m * a
    return a, c


def kernel(features, W0, b0, g0, be0, W1, b1, g1, be1, Ws, bs, gs, bes):
    pts = features[:, :, 0:2]
    X = features[:, :, 2:]

    gidx = _knn(pts).reshape(EDGES)

    W0a = W0[:, :C]
    W0b = W0[:, C:]
    U = jnp.einsum('npc,oc->npo', X, W0b).reshape(N * P, C)
    V = jnp.einsum('npc,oc->npo', X, W0a + W0b) + b0[None, None, :]

    G = jnp.take_along_axis(U, gidx[:, None], axis=0)
    y0 = (jnp.repeat(V.reshape(N * P, C), K, axis=0) - G)

    s0, ss0 = _stats(y0, 8192)
    a0, c0 = _bn_coeffs(s0, ss0, float(EDGES), g0, be0)
    y1, s1, ss1 = _mid(y0, a0, c0, W1.T, 8192)
    a1, c1 = _bn_coeffs(s1, ss1, float(EDGES), g1, be1)

    ysc = (jnp.einsum('npc,oc->npo', X, Ws) + bs[None, None, :]).reshape(N * P, C)
    ssc, sssc = _stats(ysc, 4096)
    asc, csc = _bn_coeffs(ssc, sssc, float(N * P), gs, bes)
    sc = ysc * asc[None, :] + csc[None, :]

    out = _final(y1, a1, c1, sc, 8192)
    return jnp.transpose(out.reshape(N, P, C), (0, 2, 1))


# speedup vs baseline: 4.9228x; 1.0023x over previous
import functools

import jax
import jax.numpy as jnp
from jax import lax
from jax.experimental import pallas as pl
from jax.experimental.pallas import tpu as pltpu

K = 16
N = 16
P = 2048
C = 64
EDGES = N * P * K
RT = 256
IMAX = 2147483647


def _knn_kernel(pts_r_ref, ptsT_ref, out_ref):
    n = pl.program_id(0)
    pr = pts_r_ref[0]
    pcT = ptsT_ref[0]
    dx = pr[:, 0:1] - pcT[0:1, :]
    dy = pr[:, 1:2] - pcT[1:2, :]
    dist = dx * dx + dy * dy
    col = lax.broadcasted_iota(jnp.int32, (RT, P), 1)
    bits = lax.bitcast_convert_type(dist, jnp.int32)
    keys0 = jnp.bitwise_or(jnp.bitwise_and(bits, jnp.int32(-2048)), col)

    def body(k, carry):
        keys, acc = carry
        m = keys[:, 0:128]
        for j in range(1, 16):
            m = jnp.minimum(m, keys[:, j * 128:(j + 1) * 128])
        g = jnp.min(m, axis=1, keepdims=True)
        keys = jnp.where(keys == g, IMAX, keys)
        lane = lax.broadcasted_iota(jnp.int32, (RT, K + 1), 1)
        acc = jnp.where(lane == k, jnp.bitwise_and(g, jnp.int32(2047)), acc)
        return keys, acc

    _, acc = lax.fori_loop(0, K + 1, body,
                           (keys0, jnp.zeros((RT, K + 1), jnp.int32)))
    out_ref[0] = acc[:, 1:] + n * P


def _knn(pts):
    ptsT = jnp.transpose(pts, (0, 2, 1))
    return pl.pallas_call(
        _knn_kernel,
        grid=(N, P // RT),
        in_specs=[
            pl.BlockSpec((1, RT, 2), lambda n, r: (n, r, 0)),
            pl.BlockSpec((1, 2, P), lambda n, r: (n, 0, 0)),
        ],
        out_specs=pl.BlockSpec((1, RT, K), lambda n, r: (n, r, 0)),
        out_shape=jax.ShapeDtypeStruct((N, P, K), jnp.int32),
    )(pts, ptsT)


def _stats_kernel(x_ref, s_ref, ss_ref):
    i = pl.program_id(0)
    x = x_ref[...]
    s = jnp.sum(x, axis=0, keepdims=True)
    ss = jnp.sum(x * x, axis=0, keepdims=True)

    @pl.when(i == 0)
    def _():
        s_ref[...] = s
        ss_ref[...] = ss

    @pl.when(i > 0)
    def _():
        s_ref[...] += s
        ss_ref[...] += ss


def _stats(x, rows_per_block):
    rows = x.shape[0]
    grid = rows // rows_per_block
    s, ss = pl.pallas_call(
        _stats_kernel,
        grid=(grid,),
        in_specs=[pl.BlockSpec((rows_per_block, 64), lambda i: (i, 0))],
        out_specs=[
            pl.BlockSpec((1, 64), lambda i: (0, 0)),
            pl.BlockSpec((1, 64), lambda i: (0, 0)),
        ],
        out_shape=[
            jax.ShapeDtypeStruct((1, 64), jnp.float32),
            jax.ShapeDtypeStruct((1, 64), jnp.float32),
        ],
    )(x)
    return s[0], ss[0]


def _mid_kernel(y0_ref, a0_ref, c0_ref, w1_ref, y1_ref, s_ref, ss_ref):
    i = pl.program_id(0)
    e0 = jnp.maximum(y0_ref[...] * a0_ref[...] + c0_ref[...], 0.0)
    y1 = jnp.dot(e0, w1_ref[...], preferred_element_type=jnp.float32)
    y1_ref[...] = y1
    s = jnp.sum(y1, axis=0, keepdims=True)
    ss = jnp.sum(y1 * y1, axis=0, keepdims=True)

    @pl.when(i == 0)
    def _():
        s_ref[...] = s
        ss_ref[...] = ss

    @pl.when(i > 0)
    def _():
        s_ref[...] += s
        ss_ref[...] += ss


def _mid(y0, a0, c0, w1t, rows_per_block):
    rows = y0.shape[0]
    grid = rows // rows_per_block
    y1, s, ss = pl.pallas_call(
        _mid_kernel,
        grid=(grid,),
        in_specs=[
            pl.BlockSpec((rows_per_block, 64), lambda i: (i, 0)),
            pl.BlockSpec((1, 64), lambda i: (0, 0)),
            pl.BlockSpec((1, 64), lambda i: (0, 0)),
            pl.BlockSpec((64, 64), lambda i: (0, 0)),
        ],
        out_specs=[
            pl.BlockSpec((rows_per_block, 64), lambda i: (i, 0)),
            pl.BlockSpec((1, 64), lambda i: (0, 0)),
            pl.BlockSpec((1, 64), lambda i: (0, 0)),
        ],
        out_shape=[
            jax.ShapeDtypeStruct((rows, 64), jnp.float32),
            jax.ShapeDtypeStruct((1, 64), jnp.float32),
            jax.ShapeDtypeStruct((1, 64), jnp.float32),
        ],
    )(y0, a0[None, :], c0[None, :], w1t)
    return y1, s[0], ss[0]


def _final_kernel(y1_ref, a1_ref, c1_ref, sc_ref, out_ref):
    e1 = jnp.maximum(y1_ref[...] * a1_ref[...] + c1_ref[...], 0.0)
    rows = e1.shape[0]
    h = jnp.mean(e1.reshape(rows // K, K, 64), axis=1)
    out_ref[...] = jnp.maximum(h + sc_ref[...], 0.0)


def _final(y1, a1, c1, sc, rows_per_block):
    rows = y1.shape[0]
    grid = rows // rows_per_block
    out = pl.pallas_call(
        _final_kernel,
        grid=(grid,),
        in_specs=[
            pl.BlockSpec((rows_per_block, 64), lambda i: (i, 0)),
            pl.BlockSpec((1, 64), lambda i: (0, 0)),
            pl.BlockSpec((1, 64), lambda i: (0, 0)),
            pl.BlockSpec((rows_per_block // K, 64), lambda i: (i, 0)),
        ],
        out_specs=pl.BlockSpec((rows_per_block // K, 64), lambda i: (i, 0)),
        out_shape=jax.ShapeDtypeStruct((rows // K, 64), jnp.float32),
    )(y1, a1[None, :], c1[None, :], sc)
    return out


def _bn_coeffs(s, ss, count, g, be):
    m = s / count
    v = ss / count - m * m
    a = g * lax.rsqrt(v + 1e-5)
    c = be - m * a
    return a, c


def kernel(features, W0, b0, g0, be0, W1, b1, g1, be1, Ws, bs, gs, bes):
    pts = features[:, :, 0:2]
    X = features[:, :, 2:]

    gidx = _knn(pts).reshape(EDGES)

    W0a = W0[:, :C]
    W0b = W0[:, C:]
    U = jnp.einsum('npc,oc->npo', X, W0b).reshape(N * P, C)
    V = jnp.einsum('npc,oc->npo', X, W0a + W0b) + b0[None, None, :]

    G = jnp.take_along_axis(U, gidx[:, None], axis=0)
    y0 = (jnp.repeat(V.reshape(N * P, C), K, axis=0) - G)

    s0, ss0 = _stats(y0, 8192)
    a0, c0 = _bn_coeffs(s0, ss0, float(EDGES), g0, be0)
    y1, s1, ss1 = _mid(y0, a0, c0, W1.T, 8192)
    a1, c1 = _bn_coeffs(s1, ss1, float(EDGES), g1, be1)

    ysc = (jnp.einsum('npc,oc->npo', X, Ws) + bs[None, None, :]).reshape(N * P, C)
    ssc, sssc = _stats(ysc, 4096)
    asc, csc = _bn_coeffs(ssc, sssc, float(N * P), gs, bes)
    sc = ysc * asc[None, :] + csc[None, :]

    out = _final(y1, a1, c1, sc, 8192)
    return jnp.transpose(out.reshape(N, P, C), (0, 2, 1))
